# Initial kernel scaffold; baseline (speedup 1.0000x reference)
import jax
import jax.numpy as jnp
from jax.experimental import pallas as pl


def kernel(tensor, rowperm, colperm):
    raise NotImplementedError("write your pallas kernel here")


# speedup vs baseline: 1.6582x; 1.6582x over previous
import functools

import jax
import jax.numpy as jnp
from jax import lax
from jax.experimental import pallas as pl
from jax.experimental.pallas import tpu as pltpu
from jax.experimental.pallas import tpu_sc as plsc

B, R, C = 4, 4096, 2048
NROWS = B * R
NW = 32
ROWS_PER_W = NROWS // NW
G = 16
NCHUNK = ROWS_PER_W // G
NLANE = 16
JCHUNK = C // NLANE


def _make_kernel():
    mesh = plsc.VectorSubcoreMesh(core_axis_name="c", subcore_axis_name="s")

    @functools.partial(
        pl.kernel,
        mesh=mesh,
        out_type=jax.ShapeDtypeStruct((NROWS, C), jnp.float32),
        compiler_params=pltpu.CompilerParams(needs_layout_passes=False),
        scratch_types=[
            pltpu.VMEM((NCHUNK, G), jnp.int32),
            pltpu.VMEM((C,), jnp.int32),
            pltpu.VMEM((G, C), jnp.float32),
            pltpu.VMEM((G, C), jnp.float32),
            pltpu.SemaphoreType.DMA,
        ],
    )
    def k(tens, ridx, cperm, out, idx_v, cperm_v, inbuf, outbuf, sem):
        w = lax.axis_index("s") * 2 + lax.axis_index("c")
        pltpu.sync_copy(ridx.at[w], idx_v)
        pltpu.sync_copy(cperm, cperm_v)
        row_base = w * ROWS_PER_W

        def chunk_body(c, carry):
            pltpu.async_copy(tens.at[idx_v.at[c]], inbuf, sem).wait()

            def j_body(j, carry2):
                idx16 = cperm_v[pl.ds(j * NLANE, NLANE)]
                for r in range(G):
                    ridx16 = jnp.full((NLANE,), r, dtype=jnp.int32)
                    outbuf[r, pl.ds(j * NLANE, NLANE)] = plsc.load_gather(
                        inbuf, [ridx16, idx16]
                    )
                return carry2

            lax.fori_loop(0, JCHUNK, j_body, 0)
            pltpu.sync_copy(outbuf, out.at[pl.ds(row_base + c * G, G)])
            return carry

        lax.fori_loop(0, NCHUNK, chunk_body, 0)

    return k


_sc_permute = _make_kernel()


@jax.jit
def kernel(tensor, rowperm, colperm):
    t2 = tensor.reshape(NROWS, C)
    ridx = (
        jnp.arange(B, dtype=jnp.int32)[:, None] * R
        + rowperm.astype(jnp.int32)[None, :]
    ).reshape(NW, NCHUNK, G)
    out = _sc_permute(t2, ridx, colperm.astype(jnp.int32))
    return out.reshape(B, R, C)


# speedup vs baseline: 2.0223x; 1.2195x over previous
import functools

import jax
import jax.numpy as jnp
from jax import lax
from jax.experimental import pallas as pl
from jax.experimental.pallas import tpu as pltpu
from jax.experimental.pallas import tpu_sc as plsc

B, R, C = 4, 4096, 2048
NROWS = B * R
NW = 32
ROWS_PER_W = NROWS // NW
G = 8
NCHUNK = ROWS_PER_W // G
NLANE = 16
JCHUNK = C // NLANE


def _make_kernel():
    mesh = plsc.VectorSubcoreMesh(core_axis_name="c", subcore_axis_name="s")

    @functools.partial(
        pl.kernel,
        mesh=mesh,
        out_type=jax.ShapeDtypeStruct((NROWS, C), jnp.float32),
        compiler_params=pltpu.CompilerParams(needs_layout_passes=False),
        scratch_types=[
            pltpu.VMEM((NCHUNK, G), jnp.int32),
            pltpu.VMEM((C,), jnp.int32),
            pltpu.VMEM((G, C), jnp.float32),
            pltpu.VMEM((G, C), jnp.float32),
            pltpu.VMEM((G, C), jnp.float32),
            pltpu.VMEM((G, C), jnp.float32),
            pltpu.SemaphoreType.DMA,
            pltpu.SemaphoreType.DMA,
            pltpu.SemaphoreType.DMA,
            pltpu.SemaphoreType.DMA,
        ],
    )
    def k(tens, ridx, cperm, out, idx_v, cperm_v,
          in0, in1, out0, out1, isem0, isem1, osem0, osem1):
        w = lax.axis_index("s") * 2 + lax.axis_index("c")
        pltpu.sync_copy(ridx.at[w], idx_v)
        pltpu.sync_copy(cperm, cperm_v)
        row_base = w * ROWS_PER_W

        inbufs = (in0, in1)
        isems = (isem0, isem1)
        outbufs = (out0, out1)
        osems = (osem0, osem1)

        pltpu.async_copy(tens.at[idx_v.at[0]], in0, isem0)
        pltpu.async_copy(tens.at[idx_v.at[1]], in1, isem1)

        def permute(inbuf, outbuf):
            def j_body(j, carry):
                idx16 = cperm_v[pl.ds(j * NLANE, NLANE)]
                for r in range(G):
                    ridx16 = jnp.full((NLANE,), r, dtype=jnp.int32)
                    outbuf[r, pl.ds(j * NLANE, NLANE)] = plsc.load_gather(
                        inbuf, [ridx16, idx16]
                    )
                return carry

            lax.fori_loop(0, JCHUNK, j_body, 0)

        def pair_body(t, carry):
            for p in range(2):
                c = 2 * t + p
                inbuf, outbuf = inbufs[p], outbufs[p]
                pltpu.make_async_copy(tens.at[idx_v.at[c]], inbuf, isems[p]).wait()

                @pl.when(t > 0)
                def _():
                    pltpu.make_async_copy(
                        outbuf, out.at[pl.ds(row_base + (c - 2) * G, G)], osems[p]
                    ).wait()

                permute(inbuf, outbuf)
                pltpu.async_copy(
                    outbuf, out.at[pl.ds(row_base + c * G, G)], osems[p]
                )

                @pl.when(c + 2 < NCHUNK)
                def _():
                    pltpu.async_copy(tens.at[idx_v.at[c + 2]], inbuf, isems[p])
            return carry

        lax.fori_loop(0, NCHUNK // 2, pair_body, 0)

        for p in range(2):
            c_last = NCHUNK - 2 + p
            pltpu.make_async_copy(
                outbufs[p], out.at[pl.ds(row_base + c_last * G, G)], osems[p]
            ).wait()

    return k


_sc_permute = _make_kernel()


@jax.jit
def kernel(tensor, rowperm, colperm):
    t2 = tensor.reshape(NROWS, C)
    ridx = (
        jnp.arange(B, dtype=jnp.int32)[:, None] * R
        + rowperm.astype(jnp.int32)[None, :]
    ).reshape(NW, NCHUNK, G)
    out = _sc_permute(t2, ridx, colperm.astype(jnp.int32))
    return out.reshape(B, R, C)


# speedup vs baseline: 6.0495x; 2.9914x over previous
import functools

import jax
import jax.numpy as jnp
from jax import lax
from jax.experimental import pallas as pl
from jax.experimental.pallas import tpu as pltpu
from jax.experimental.pallas import tpu_sc as plsc

B, R, C = 4, 4096, 2048
NROWS = B * R
NW = 32
ROWS_PER_W = NROWS // NW
G = 8
NCHUNK = ROWS_PER_W // G
NLANE = 16
JCHUNK = C // NLANE


def _make_kernel():
    mesh = plsc.VectorSubcoreMesh(core_axis_name="c", subcore_axis_name="s")

    @functools.partial(
        pl.kernel,
        mesh=mesh,
        out_type=jax.ShapeDtypeStruct((NROWS, C), jnp.float32),
        compiler_params=pltpu.CompilerParams(needs_layout_passes=False),
        scratch_types=[
            pltpu.VMEM((NCHUNK, G), jnp.int32),
            pltpu.VMEM((C,), jnp.int32),
            pltpu.VMEM((G, C), jnp.float32),
            pltpu.VMEM((G, C), jnp.float32),
            pltpu.VMEM((G, C), jnp.float32),
            pltpu.VMEM((G, C), jnp.float32),
            pltpu.SemaphoreType.DMA,
            pltpu.SemaphoreType.DMA,
            pltpu.SemaphoreType.DMA,
            pltpu.SemaphoreType.DMA,
        ],
    )
    def k(tens, ridx, cperm, out, idx_v, cperm_v,
          in0, in1, out0, out1, isem0, isem1, osem0, osem1):
        w = lax.axis_index("s") * 2 + lax.axis_index("c")
        pltpu.sync_copy(ridx.at[w], idx_v)
        pltpu.sync_copy(cperm, cperm_v)
        row_base = w * ROWS_PER_W

        inbufs = (in0, in1)
        isems = (isem0, isem1)
        outbufs = (out0, out1)
        osems = (osem0, osem1)

        pltpu.async_copy(tens.at[idx_v.at[0]], in0, isem0)
        pltpu.async_copy(tens.at[idx_v.at[1]], in1, isem1)

        def permute(inbuf, outbuf):
            @plsc.parallel_loop(0, JCHUNK, unroll=4)
            def _(j):
                idx16 = cperm_v[pl.ds(j * NLANE, NLANE)]
                for r in range(G):
                    ridx16 = jnp.full((NLANE,), r, dtype=jnp.int32)
                    outbuf[r, pl.ds(j * NLANE, NLANE)] = plsc.load_gather(
                        inbuf, [ridx16, idx16]
                    )

        def pair_body(t, carry):
            for p in range(2):
                c = 2 * t + p
                inbuf, outbuf = inbufs[p], outbufs[p]
                pltpu.make_async_copy(tens.at[idx_v.at[c]], inbuf, isems[p]).wait()

                @pl.when(t > 0)
                def _():
                    pltpu.make_async_copy(
                        outbuf, out.at[pl.ds(row_base + (c - 2) * G, G)], osems[p]
                    ).wait()

                permute(inbuf, outbuf)
                pltpu.async_copy(
                    outbuf, out.at[pl.ds(row_base + c * G, G)], osems[p]
                )

                @pl.when(c + 2 < NCHUNK)
                def _():
                    pltpu.async_copy(tens.at[idx_v.at[c + 2]], inbuf, isems[p])
            return carry

        lax.fori_loop(0, NCHUNK // 2, pair_body, 0)

        for p in range(2):
            c_last = NCHUNK - 2 + p
            pltpu.make_async_copy(
                outbufs[p], out.at[pl.ds(row_base + c_last * G, G)], osems[p]
            ).wait()

    return k


_sc_permute = _make_kernel()


@jax.jit
def kernel(tensor, rowperm, colperm):
    t2 = tensor.reshape(NROWS, C)
    ridx = (
        jnp.arange(B, dtype=jnp.int32)[:, None] * R
        + rowperm.astype(jnp.int32)[None, :]
    ).reshape(NW, NCHUNK, G)
    out = _sc_permute(t2, ridx, colperm.astype(jnp.int32))
    return out.reshape(B, R, C)
